# baseline (speedup 1.0000x reference)
import functools

import numpy as np

import jax
import jax.numpy as jnp
from jax import lax
from jax.experimental import pallas as pl
from jax.experimental.pallas import tpu as pltpu
from jax.experimental.pallas import tpu_sc as plsc

N = 10000
R = 10240
CH = 128
NC = 2
NS = 16
NW = NC * NS
K = 128
CPT = 80
SHIFT = 14
EPAD = NW * CPT * K
ZPT = R // NS
BR = 1024
GRID = R // BR

@functools.cache
def _mesh():
  return plsc.VectorSubcoreMesh(
      core_axis_name="c", subcore_axis_name="s", num_cores=NC, num_subcores=NS)


def _fill(ref, value):
  @pl.loop(0, K)
  def _row(r):
    for g in range(CH // 16):
      ref[r, pl.ds(g * 16, 16)] = jnp.full((16,), value, jnp.float32)


def _zero_my_slice(acc, zbuf, s):
  @pl.loop(0, ZPT // K)
  def _blk(t):
    pltpu.sync_copy(zbuf, acc.at[pl.ds(s * ZPT + t * K, K)])


def _deg_body(packed_hbm, out_hbm, packedv, onesv, dstb, acc, sem):
  c = lax.axis_index("c")
  s = lax.axis_index("s")
  wid = c * NS + s
  pltpu.sync_copy(packed_hbm.at[pl.ds(wid * CPT, CPT)], packedv)
  _fill(onesv, 0.0)
  _zero_my_slice(acc, onesv, s)
  _fill(onesv, 1.0)
  plsc.subcore_barrier()

  @pl.loop(0, CPT)
  def _chunk(j):
    for i in range(K // 16):
      p = packedv[j, pl.ds(i * 16, 16)]
      dstb[pl.ds(i * 16, 16)] = lax.shift_right_logical(p, SHIFT)
    pltpu.sync_copy(onesv, acc.at[dstb], add=True)

  plsc.subcore_barrier()
  pltpu.sync_copy(acc.at[pl.ds(s * ZPT, ZPT)],
                  out_hbm.at[pl.ds(c * R + s * ZPT, ZPT)])


@functools.cache
def _deg_call():
  return pl.kernel(
      _deg_body,
      out_type=jax.ShapeDtypeStruct((NC * R, CH), jnp.float32),
      mesh=_mesh(),
      scratch_types=[
          pltpu.VMEM((CPT, K), jnp.int32),
          pltpu.VMEM((K, CH), jnp.float32),
          pltpu.VMEM((K,), jnp.int32),
          pltpu.VMEM_SHARED((R, CH), jnp.float32),
          pltpu.SemaphoreType.DMA,
      ],
  )


def _unpack(packedv, j, srcb, dstb):
  for i in range(K // 16):
    p = packedv[j, pl.ds(i * 16, 16)]
    srcb[pl.ds(i * 16, 16)] = lax.bitwise_and(p, (1 << SHIFT) - 1)
    dstb[pl.ds(i * 16, 16)] = lax.shift_right_logical(p, SHIFT)


def _prop_body(y_hbm, packed_hbm, out_hbm,
               packedv, srcb0, srcb1, dstb0, dstb1, rows0, rows1, acc,
               sem0, sem1):
  c = lax.axis_index("c")
  s = lax.axis_index("s")
  wid = c * NS + s
  pltpu.sync_copy(packed_hbm.at[pl.ds(wid * CPT, CPT)], packedv)
  srcb = (srcb0, srcb1)
  dstb = (dstb0, dstb1)
  rows = (rows0, rows1)
  sems = (sem0, sem1)
  _fill(rows0, 0.0)
  _zero_my_slice(acc, rows0, s)
  _unpack(packedv, 0, srcb0, dstb0)
  pltpu.async_copy(y_hbm.at[srcb0], rows0, sem0)
  plsc.subcore_barrier()

  @pl.loop(0, CPT, step=2)
  def _chunk(j):
    for b in range(2):
      jj = j + b
      nj = jnp.minimum(jj + 1, CPT - 1)
      _unpack(packedv, nj, srcb[1 - b], dstb[1 - b])
      pltpu.async_copy(y_hbm.at[srcb[1 - b]], rows[1 - b], sems[1 - b])
      pltpu.make_async_copy(y_hbm.at[srcb[b]], rows[b], sems[b]).wait()
      pltpu.sync_copy(rows[b], acc.at[dstb[b]], add=True)

  pltpu.make_async_copy(y_hbm.at[srcb0], rows0, sem0).wait()
  plsc.subcore_barrier()
  pltpu.sync_copy(acc.at[pl.ds(s * ZPT, ZPT)],
                  out_hbm.at[pl.ds(c * R + s * ZPT, ZPT)])


@functools.cache
def _prop_call():
  return pl.kernel(
      _prop_body,
      out_type=jax.ShapeDtypeStruct((NC * R, CH), jnp.float32),
      mesh=_mesh(),
      scratch_types=[
          pltpu.VMEM((CPT, K), jnp.int32),
          pltpu.VMEM((K,), jnp.int32),
          pltpu.VMEM((K,), jnp.int32),
          pltpu.VMEM((K,), jnp.int32),
          pltpu.VMEM((K,), jnp.int32),
          pltpu.VMEM((K, CH), jnp.float32),
          pltpu.VMEM((K, CH), jnp.float32),
          pltpu.VMEM_SHARED((R, CH), jnp.float32),
          pltpu.SemaphoreType.DMA,
          pltpu.SemaphoreType.DMA,
      ],
  )


def _dinv(d_ref):
  dsum = d_ref[0, :, 0:1] + d_ref[1, :, 0:1]
  return lax.rsqrt(1.0 + dsum)


def _y1_body(x_ref, w_ref, d_ref, y_ref):
  y_ref[...] = jnp.dot(x_ref[...], w_ref[...],
                       preferred_element_type=jnp.float32) * _dinv(d_ref)


_y1_call = pl.pallas_call(
    _y1_body,
    grid=(GRID,),
    in_specs=[
        pl.BlockSpec((BR, CH), lambda i: (i, 0)),
        pl.BlockSpec((CH, CH), lambda i: (0, 0)),
        pl.BlockSpec((2, BR, 16), lambda i: (0, i, 0)),
    ],
    out_specs=pl.BlockSpec((BR, CH), lambda i: (i, 0)),
    out_shape=jax.ShapeDtypeStruct((R, CH), jnp.float32),
)


def _mid_body(s_ref, y1_ref, d_ref, w_ref, b_ref, y2_ref):
  dinv = _dinv(d_ref)
  h = (s_ref[0] + s_ref[1] + y1_ref[...]) * dinv + b_ref[...]
  h = jnp.maximum(h, 0.0)
  y2_ref[...] = jnp.dot(h, w_ref[...],
                        preferred_element_type=jnp.float32) * dinv


_mid_call = pl.pallas_call(
    _mid_body,
    grid=(GRID,),
    in_specs=[
        pl.BlockSpec((2, BR, CH), lambda i: (0, i, 0)),
        pl.BlockSpec((BR, CH), lambda i: (i, 0)),
        pl.BlockSpec((2, BR, 16), lambda i: (0, i, 0)),
        pl.BlockSpec((CH, CH), lambda i: (0, 0)),
        pl.BlockSpec((1, CH), lambda i: (0, 0)),
    ],
    out_specs=pl.BlockSpec((BR, CH), lambda i: (i, 0)),
    out_shape=jax.ShapeDtypeStruct((R, CH), jnp.float32),
)


def _out_body(s_ref, y2_ref, d_ref, b_ref, mu_ref, ls_ref):
  o = (s_ref[0] + s_ref[1] + y2_ref[...]) * _dinv(d_ref) + b_ref[...]
  mu_ref[...] = o[:, :CH // 2]
  ls_ref[...] = o[:, CH // 2:]


_out_call = pl.pallas_call(
    _out_body,
    grid=(GRID,),
    in_specs=[
        pl.BlockSpec((2, BR, CH), lambda i: (0, i, 0)),
        pl.BlockSpec((BR, CH), lambda i: (i, 0)),
        pl.BlockSpec((2, BR, 16), lambda i: (0, i, 0)),
        pl.BlockSpec((1, CH), lambda i: (0, 0)),
    ],
    out_specs=[
        pl.BlockSpec((BR, CH // 2), lambda i: (i, 0)),
        pl.BlockSpec((BR, CH // 2), lambda i: (i, 0)),
    ],
    out_shape=[
        jax.ShapeDtypeStruct((N, CH // 2), jnp.float32),
        jax.ShapeDtypeStruct((N, CH // 2), jnp.float32),
    ],
)


def _pack_body(e0_ref, e1_ref, pad_ref, o_ref):
  base = pl.program_id(0) * (EPAD // K // GRID)
  row = base + lax.broadcasted_iota(jnp.int32, (EPAD // K // GRID, 1), 0)
  packed = e0_ref[...] + e1_ref[...] * (1 << SHIFT)
  o_ref[...] = jnp.where(row < N_EDGE_ROWS, packed, pad_ref[...])


N_EDGE_ROWS = 2500


@functools.cache
def _pack_call():
  blk = EPAD // K // GRID
  return pl.pallas_call(
      _pack_body,
      grid=(GRID,),
      in_specs=[
          pl.BlockSpec((blk, K), lambda i: (i, 0)),
          pl.BlockSpec((blk, K), lambda i: (i, 0)),
          pl.BlockSpec((blk, K), lambda i: (i, 0)),
      ],
      out_specs=pl.BlockSpec((blk, K), lambda i: (i, 0)),
      out_shape=jax.ShapeDtypeStruct((NW * CPT, K), jnp.int32),
  )


@functools.cache
def _pad_const():
  npad = EPAD - N_EDGE_ROWS * K
  fill = np.arange(npad)
  pad = (fill % N + (N + fill % (R - N)) * (1 << SHIFT)).astype(np.int32)
  full = np.zeros((NW * CPT, K), np.int32)
  full[N_EDGE_ROWS:] = pad.reshape(-1, K)
  return jnp.asarray(full)


@jax.jit
def _run(x, edge_index, W1, b1, Wmu, bmu, Wls, bls):
  Wml = jnp.concatenate([Wmu, Wls], axis=1)
  bml = jnp.concatenate([bmu, bls]).reshape(1, CH)
  b1r = b1.reshape(1, CH)

  e0 = edge_index[0].reshape(N_EDGE_ROWS, K)
  e1 = edge_index[1].reshape(N_EDGE_ROWS, K)
  packed = _pack_call()(e0, e1, _pad_const())

  deg = _deg_call()(packed).reshape(NC, R, CH)[:, :, :16]
  y1 = _y1_call(x, W1, deg)
  s1 = _prop_call()(y1, packed).reshape(NC, R, CH)
  y2 = _mid_call(s1, y1, deg, Wml, b1r)
  s2 = _prop_call()(y2, packed).reshape(NC, R, CH)
  mu, logstd = _out_call(s2, y2, deg, bml)
  return mu, logstd


def kernel(x, edge_index, W1, b1, Wmu, bmu, Wls, bls):
  return _run(x, edge_index, W1, b1, Wmu, bmu, Wls, bls)


# speedup vs baseline: 1.0159x; 1.0159x over previous
import functools

import numpy as np

import jax
import jax.numpy as jnp
from jax import lax
from jax.experimental import pallas as pl
from jax.experimental.pallas import tpu as pltpu
from jax.experimental.pallas import tpu_sc as plsc

N = 10000
R = 10240
CH = 128
NC = 2
NS = 16
NW = NC * NS
K = 128
CPT = 80
SHIFT = 14
EPAD = NW * CPT * K
ZPT = R // NS
BR = 1024
GRID = R // BR

@functools.cache
def _mesh():
  return plsc.VectorSubcoreMesh(
      core_axis_name="c", subcore_axis_name="s", num_cores=NC, num_subcores=NS)


def _fill(ref, value):
  @pl.loop(0, K)
  def _row(r):
    for g in range(CH // 16):
      ref[r, pl.ds(g * 16, 16)] = jnp.full((16,), value, jnp.float32)


def _zero_my_slice(acc, zbuf, s):
  @pl.loop(0, ZPT // K)
  def _blk(t):
    pltpu.sync_copy(zbuf, acc.at[pl.ds(s * ZPT + t * K, K)])


def _deg_body(packed_hbm, out_hbm, packedv, onesv, dstb, acc, sem):
  c = lax.axis_index("c")
  s = lax.axis_index("s")
  wid = c * NS + s
  pltpu.sync_copy(packed_hbm.at[pl.ds(wid * CPT, CPT)], packedv)
  _fill(onesv, 0.0)
  _zero_my_slice(acc, onesv, s)
  _fill(onesv, 1.0)
  plsc.subcore_barrier()

  @pl.loop(0, CPT)
  def _chunk(j):
    for i in range(K // 16):
      p = packedv[j, pl.ds(i * 16, 16)]
      dstb[pl.ds(i * 16, 16)] = lax.shift_right_logical(p, SHIFT)
    pltpu.sync_copy(onesv, acc.at[dstb], add=True)

  plsc.subcore_barrier()
  pltpu.sync_copy(acc.at[pl.ds(s * ZPT, ZPT)],
                  out_hbm.at[pl.ds(c * R + s * ZPT, ZPT)])


@functools.cache
def _deg_call():
  return pl.kernel(
      _deg_body,
      out_type=jax.ShapeDtypeStruct((NC * R, CH), jnp.float32),
      mesh=_mesh(),
      scratch_types=[
          pltpu.VMEM((CPT, K), jnp.int32),
          pltpu.VMEM((K, CH), jnp.float32),
          pltpu.VMEM((K,), jnp.int32),
          pltpu.VMEM_SHARED((R, CH), jnp.float32),
          pltpu.SemaphoreType.DMA,
      ],
  )


def _unpack(packedv, j, srcb, dstb):
  for i in range(K // 16):
    p = packedv[j, pl.ds(i * 16, 16)]
    srcb[pl.ds(i * 16, 16)] = lax.bitwise_and(p, (1 << SHIFT) - 1)
    dstb[pl.ds(i * 16, 16)] = lax.shift_right_logical(p, SHIFT)


def _prop_body(y_hbm, packed_hbm, out_hbm,
               packedv, srcb0, srcb1, dstb0, dstb1, rows0, rows1, acc,
               sem0, sem1):
  c = lax.axis_index("c")
  s = lax.axis_index("s")
  wid = c * NS + s
  pltpu.sync_copy(packed_hbm.at[pl.ds(wid * CPT, CPT)], packedv)
  srcb = (srcb0, srcb1)
  dstb = (dstb0, dstb1)
  rows = (rows0, rows1)
  sems = (sem0, sem1)
  _fill(rows0, 0.0)
  _zero_my_slice(acc, rows0, s)
  _unpack(packedv, 0, srcb0, dstb0)
  pltpu.async_copy(y_hbm.at[srcb0], rows0, sem0)
  plsc.subcore_barrier()

  @pl.loop(0, CPT, step=2)
  def _chunk(j):
    for b in range(2):
      jj = j + b
      nj = jnp.minimum(jj + 1, CPT - 1)
      _unpack(packedv, nj, srcb[1 - b], dstb[1 - b])
      pltpu.async_copy(y_hbm.at[srcb[1 - b]], rows[1 - b], sems[1 - b])
      pltpu.make_async_copy(y_hbm.at[srcb[b]], rows[b], sems[b]).wait()
      pltpu.sync_copy(rows[b], acc.at[dstb[b]], add=True)

  pltpu.make_async_copy(y_hbm.at[srcb0], rows0, sem0).wait()
  plsc.subcore_barrier()
  pltpu.sync_copy(acc.at[pl.ds(s * ZPT, ZPT)],
                  out_hbm.at[pl.ds(c * R + s * ZPT, ZPT)])


@functools.cache
def _prop_call():
  return pl.kernel(
      _prop_body,
      out_type=jax.ShapeDtypeStruct((NC * R, CH), jnp.float32),
      mesh=_mesh(),
      scratch_types=[
          pltpu.VMEM((CPT, K), jnp.int32),
          pltpu.VMEM((K,), jnp.int32),
          pltpu.VMEM((K,), jnp.int32),
          pltpu.VMEM((K,), jnp.int32),
          pltpu.VMEM((K,), jnp.int32),
          pltpu.VMEM((K, CH), jnp.float32),
          pltpu.VMEM((K, CH), jnp.float32),
          pltpu.VMEM_SHARED((R, CH), jnp.float32),
          pltpu.SemaphoreType.DMA,
          pltpu.SemaphoreType.DMA,
      ],
  )


def _dinv(d_ref):
  dsum = d_ref[0, :, 0:1] + d_ref[1, :, 0:1]
  return lax.rsqrt(1.0 + dsum)


def _y1_body(x_ref, w_ref, d_ref, y_ref):
  y_ref[...] = jnp.dot(x_ref[...], w_ref[...],
                       preferred_element_type=jnp.float32) * _dinv(d_ref)


_y1_call = pl.pallas_call(
    _y1_body,
    grid=(GRID,),
    in_specs=[
        pl.BlockSpec((BR, CH), lambda i: (i, 0)),
        pl.BlockSpec((CH, CH), lambda i: (0, 0)),
        pl.BlockSpec((2, BR, 16), lambda i: (0, i, 0)),
    ],
    out_specs=pl.BlockSpec((BR, CH), lambda i: (i, 0)),
    out_shape=jax.ShapeDtypeStruct((R, CH), jnp.float32),
)


def _mid_body(s_ref, y1_ref, d_ref, w_ref, b_ref, y2_ref):
  dinv = _dinv(d_ref)
  h = (s_ref[0] + s_ref[1] + y1_ref[...]) * dinv + b_ref[...]
  h = jnp.maximum(h, 0.0)
  y2_ref[...] = jnp.dot(h, w_ref[...],
                        preferred_element_type=jnp.float32) * dinv


_mid_call = pl.pallas_call(
    _mid_body,
    grid=(GRID,),
    in_specs=[
        pl.BlockSpec((2, BR, CH), lambda i: (0, i, 0)),
        pl.BlockSpec((BR, CH), lambda i: (i, 0)),
        pl.BlockSpec((2, BR, 16), lambda i: (0, i, 0)),
        pl.BlockSpec((CH, CH), lambda i: (0, 0)),
        pl.BlockSpec((1, CH), lambda i: (0, 0)),
    ],
    out_specs=pl.BlockSpec((BR, CH), lambda i: (i, 0)),
    out_shape=jax.ShapeDtypeStruct((R, CH), jnp.float32),
)


def _out_body(s_ref, y2_ref, d_ref, b_ref, mu_ref, ls_ref):
  o = (s_ref[0] + s_ref[1] + y2_ref[...]) * _dinv(d_ref) + b_ref[...]
  mu_ref[...] = o[:, :CH // 2]
  ls_ref[...] = o[:, CH // 2:]


_out_call = pl.pallas_call(
    _out_body,
    grid=(GRID,),
    in_specs=[
        pl.BlockSpec((2, BR, CH), lambda i: (0, i, 0)),
        pl.BlockSpec((BR, CH), lambda i: (i, 0)),
        pl.BlockSpec((2, BR, 16), lambda i: (0, i, 0)),
        pl.BlockSpec((1, CH), lambda i: (0, 0)),
    ],
    out_specs=[
        pl.BlockSpec((BR, CH // 2), lambda i: (i, 0)),
        pl.BlockSpec((BR, CH // 2), lambda i: (i, 0)),
    ],
    out_shape=[
        jax.ShapeDtypeStruct((N, CH // 2), jnp.float32),
        jax.ShapeDtypeStruct((N, CH // 2), jnp.float32),
    ],
)


@jax.jit
def _run(x, edge_index, W1, b1, Wmu, bmu, Wls, bls):
  Wml = jnp.concatenate([Wmu, Wls], axis=1)
  bml = jnp.concatenate([bmu, bls]).reshape(1, CH)
  b1r = b1.reshape(1, CH)

  npad = EPAD - edge_index.shape[1]
  fill = np.arange(npad)
  pad_packed = jnp.asarray(
      (fill % N + (N + fill % (R - N)) * (1 << SHIFT)).astype(np.int32))
  packed = jnp.concatenate(
      [edge_index[0] + edge_index[1] * (1 << SHIFT), pad_packed]
  ).reshape(NW * CPT, K)

  deg = _deg_call()(packed).reshape(NC, R, CH)[:, :, :16]
  y1 = _y1_call(x, W1, deg)
  s1 = _prop_call()(y1, packed).reshape(NC, R, CH)
  y2 = _mid_call(s1, y1, deg, Wml, b1r)
  s2 = _prop_call()(y2, packed).reshape(NC, R, CH)
  mu, logstd = _out_call(s2, y2, deg, bml)
  return mu, logstd


def kernel(x, edge_index, W1, b1, Wmu, bmu, Wls, bls):
  return _run(x, edge_index, W1, b1, Wmu, bmu, Wls, bls)


# speedup vs baseline: 1.0262x; 1.0102x over previous
import functools

import numpy as np

import jax
import jax.numpy as jnp
from jax import lax
from jax.experimental import pallas as pl
from jax.experimental.pallas import tpu as pltpu
from jax.experimental.pallas import tpu_sc as plsc

N = 10000
R = 10240
CH = 128
NC = 2
NS = 16
NW = NC * NS
K = 128
CPT = 80
SHIFT = 14
EPAD = NW * CPT * K
ZPT = R // NS
BR = 1024
GRID = R // BR

@functools.cache
def _mesh():
  return plsc.VectorSubcoreMesh(
      core_axis_name="c", subcore_axis_name="s", num_cores=NC, num_subcores=NS)


def _fill(ref, value):
  @pl.loop(0, K)
  def _row(r):
    for g in range(CH // 16):
      ref[r, pl.ds(g * 16, 16)] = jnp.full((16,), value, jnp.float32)


def _zero_my_slice(acc, zbuf, s):
  @pl.loop(0, ZPT // K)
  def _blk(t):
    pltpu.sync_copy(zbuf, acc.at[pl.ds(s * ZPT + t * K, K)])


def _deg_body(packed_hbm, out_hbm, packedv, onesv, dstb, acc, sem):
  c = lax.axis_index("c")
  s = lax.axis_index("s")
  wid = c * NS + s
  cp = pltpu.async_copy(packed_hbm.at[pl.ds(wid * CPT, CPT)], packedv, sem)
  _fill(onesv, 0.0)
  _zero_my_slice(acc, onesv, s)
  _fill(onesv, 1.0)
  cp.wait()
  plsc.subcore_barrier()

  @pl.loop(0, CPT)
  def _chunk(j):
    for i in range(K // 16):
      p = packedv[j, pl.ds(i * 16, 16)]
      dstb[pl.ds(i * 16, 16)] = lax.shift_right_logical(p, SHIFT)
    pltpu.sync_copy(onesv, acc.at[dstb], add=True)

  plsc.subcore_barrier()
  pltpu.sync_copy(acc.at[pl.ds(s * ZPT, ZPT)],
                  out_hbm.at[pl.ds(c * R + s * ZPT, ZPT)])


@functools.cache
def _deg_call():
  return pl.kernel(
      _deg_body,
      out_type=jax.ShapeDtypeStruct((NC * R, CH), jnp.float32),
      mesh=_mesh(),
      scratch_types=[
          pltpu.VMEM((CPT, K), jnp.int32),
          pltpu.VMEM((K, CH), jnp.float32),
          pltpu.VMEM((K,), jnp.int32),
          pltpu.VMEM_SHARED((R, CH), jnp.float32),
          pltpu.SemaphoreType.DMA,
      ],
  )


def _unpack(packedv, j, srcb, dstb):
  for i in range(K // 16):
    p = packedv[j, pl.ds(i * 16, 16)]
    srcb[pl.ds(i * 16, 16)] = lax.bitwise_and(p, (1 << SHIFT) - 1)
    dstb[pl.ds(i * 16, 16)] = lax.shift_right_logical(p, SHIFT)


def _prop_body(y_hbm, packed_hbm, out_hbm,
               packedv, srcb0, srcb1, dstb0, dstb1, rows0, rows1, acc,
               sem0, sem1):
  c = lax.axis_index("c")
  s = lax.axis_index("s")
  wid = c * NS + s
  cp = pltpu.async_copy(packed_hbm.at[pl.ds(wid * CPT, CPT)], packedv, sem1)
  srcb = (srcb0, srcb1)
  dstb = (dstb0, dstb1)
  rows = (rows0, rows1)
  sems = (sem0, sem1)
  _fill(rows0, 0.0)
  _zero_my_slice(acc, rows0, s)
  cp.wait()
  _unpack(packedv, 0, srcb0, dstb0)
  pltpu.async_copy(y_hbm.at[srcb0], rows0, sem0)
  plsc.subcore_barrier()

  @pl.loop(0, CPT, step=2)
  def _chunk(j):
    for b in range(2):
      jj = j + b
      nj = jnp.minimum(jj + 1, CPT - 1)
      _unpack(packedv, nj, srcb[1 - b], dstb[1 - b])
      pltpu.async_copy(y_hbm.at[srcb[1 - b]], rows[1 - b], sems[1 - b])
      pltpu.make_async_copy(y_hbm.at[srcb[b]], rows[b], sems[b]).wait()
      pltpu.sync_copy(rows[b], acc.at[dstb[b]], add=True)

  pltpu.make_async_copy(y_hbm.at[srcb0], rows0, sem0).wait()
  plsc.subcore_barrier()
  pltpu.sync_copy(acc.at[pl.ds(s * ZPT, ZPT)],
                  out_hbm.at[pl.ds(c * R + s * ZPT, ZPT)])


@functools.cache
def _prop_call():
  return pl.kernel(
      _prop_body,
      out_type=jax.ShapeDtypeStruct((NC * R, CH), jnp.float32),
      mesh=_mesh(),
      scratch_types=[
          pltpu.VMEM((CPT, K), jnp.int32),
          pltpu.VMEM((K,), jnp.int32),
          pltpu.VMEM((K,), jnp.int32),
          pltpu.VMEM((K,), jnp.int32),
          pltpu.VMEM((K,), jnp.int32),
          pltpu.VMEM((K, CH), jnp.float32),
          pltpu.VMEM((K, CH), jnp.float32),
          pltpu.VMEM_SHARED((R, CH), jnp.float32),
          pltpu.SemaphoreType.DMA,
          pltpu.SemaphoreType.DMA,
      ],
  )


def _dinv(d_ref):
  dsum = d_ref[0, :, 0:1] + d_ref[1, :, 0:1]
  return lax.rsqrt(1.0 + dsum)


def _y1_body(x_ref, w_ref, d_ref, y_ref):
  y_ref[...] = jnp.dot(x_ref[...], w_ref[...],
                       preferred_element_type=jnp.float32) * _dinv(d_ref)


_y1_call = pl.pallas_call(
    _y1_body,
    grid=(GRID,),
    in_specs=[
        pl.BlockSpec((BR, CH), lambda i: (i, 0)),
        pl.BlockSpec((CH, CH), lambda i: (0, 0)),
        pl.BlockSpec((2, BR, 16), lambda i: (0, i, 0)),
    ],
    out_specs=pl.BlockSpec((BR, CH), lambda i: (i, 0)),
    out_shape=jax.ShapeDtypeStruct((R, CH), jnp.float32),
)


def _mid_body(s_ref, y1_ref, d_ref, w_ref, b_ref, y2_ref):
  dinv = _dinv(d_ref)
  h = (s_ref[0] + s_ref[1] + y1_ref[...]) * dinv + b_ref[...]
  h = jnp.maximum(h, 0.0)
  y2_ref[...] = jnp.dot(h, w_ref[...],
                        preferred_element_type=jnp.float32) * dinv


_mid_call = pl.pallas_call(
    _mid_body,
    grid=(GRID,),
    in_specs=[
        pl.BlockSpec((2, BR, CH), lambda i: (0, i, 0)),
        pl.BlockSpec((BR, CH), lambda i: (i, 0)),
        pl.BlockSpec((2, BR, 16), lambda i: (0, i, 0)),
        pl.BlockSpec((CH, CH), lambda i: (0, 0)),
        pl.BlockSpec((1, CH), lambda i: (0, 0)),
    ],
    out_specs=pl.BlockSpec((BR, CH), lambda i: (i, 0)),
    out_shape=jax.ShapeDtypeStruct((R, CH), jnp.float32),
)


def _out_body(s_ref, y2_ref, d_ref, b_ref, mu_ref, ls_ref):
  o = (s_ref[0] + s_ref[1] + y2_ref[...]) * _dinv(d_ref) + b_ref[...]
  mu_ref[...] = o[:, :CH // 2]
  ls_ref[...] = o[:, CH // 2:]


_out_call = pl.pallas_call(
    _out_body,
    grid=(GRID,),
    in_specs=[
        pl.BlockSpec((2, BR, CH), lambda i: (0, i, 0)),
        pl.BlockSpec((BR, CH), lambda i: (i, 0)),
        pl.BlockSpec((2, BR, 16), lambda i: (0, i, 0)),
        pl.BlockSpec((1, CH), lambda i: (0, 0)),
    ],
    out_specs=[
        pl.BlockSpec((BR, CH // 2), lambda i: (i, 0)),
        pl.BlockSpec((BR, CH // 2), lambda i: (i, 0)),
    ],
    out_shape=[
        jax.ShapeDtypeStruct((N, CH // 2), jnp.float32),
        jax.ShapeDtypeStruct((N, CH // 2), jnp.float32),
    ],
)


@jax.jit
def _run(x, edge_index, W1, b1, Wmu, bmu, Wls, bls):
  Wml = jnp.concatenate([Wmu, Wls], axis=1)
  bml = jnp.concatenate([bmu, bls]).reshape(1, CH)
  b1r = b1.reshape(1, CH)

  npad = EPAD - edge_index.shape[1]
  fill = np.arange(npad)
  pad_packed = jnp.asarray(
      (fill % N + (N + fill % (R - N)) * (1 << SHIFT)).astype(np.int32))
  packed = jnp.concatenate(
      [edge_index[0] + edge_index[1] * (1 << SHIFT), pad_packed]
  ).reshape(NW * CPT, K)

  deg = _deg_call()(packed).reshape(NC, R, CH)[:, :, :16]
  y1 = _y1_call(x, W1, deg)
  s1 = _prop_call()(y1, packed).reshape(NC, R, CH)
  y2 = _mid_call(s1, y1, deg, Wml, b1r)
  s2 = _prop_call()(y2, packed).reshape(NC, R, CH)
  mu, logstd = _out_call(s2, y2, deg, bml)
  return mu, logstd


def kernel(x, edge_index, W1, b1, Wmu, bmu, Wls, bls):
  return _run(x, edge_index, W1, b1, Wmu, bmu, Wls, bls)
